# baseline (speedup 1.0000x reference)
import functools

import jax
import jax.numpy as jnp
from jax import lax
from jax.experimental import pallas as pl
from jax.experimental.pallas import tpu as pltpu
from jax.experimental.pallas import tpu_sc as plsc

N = 10000
E = 320000
D = 128
NC, NS = 2, 16
NW = NC * NS
EPW = E // NW
CHUNK = 104
NCHUNK = 98
EPW_PAD = NCHUNK * CHUNK
NCH2 = NCHUNK // 2
NPAD = N + 8
RPS = 624
RPS_LAST = N - (NS - 1) * RPS


def _sc_aggregate(xp, src_flat, dst_r, zeros):
    mesh = plsc.VectorSubcoreMesh(core_axis_name="c", subcore_axis_name="s")

    @functools.partial(
        pl.kernel,
        out_type=jax.ShapeDtypeStruct((NC, N, D), jnp.float32),
        mesh=mesh,
        scratch_types=[
            pltpu.VMEM((EPW_PAD,), jnp.int32),
            pltpu.VMEM((NCHUNK, CHUNK), jnp.int32),
            pltpu.VMEM((CHUNK, D), jnp.float32),
            pltpu.VMEM((CHUNK, D), jnp.float32),
            pltpu.VMEM_SHARED((N, D), jnp.float32),
            pltpu.SemaphoreType.DMA,
            pltpu.SemaphoreType.DMA,
        ],
    )
    def k(x_hbm, src_hbm, dst_hbm, z_hbm, out_hbm, src_v, dst_v,
          rows0_v, rows1_v, agg_sh, sem0, sem1):
        c = lax.axis_index("c")
        s = lax.axis_index("s")
        wid = c * NS + s
        pltpu.sync_copy(src_hbm.at[pl.ds(wid * EPW_PAD, EPW_PAD)], src_v)
        pltpu.sync_copy(dst_hbm.at[wid], dst_v)
        r0 = s * RPS

        @pl.when(s < NS - 1)
        def _():
            pltpu.sync_copy(z_hbm.at[pl.ds(0, RPS)], agg_sh.at[pl.ds(r0, RPS)])

        @pl.when(s == NS - 1)
        def _():
            pltpu.sync_copy(
                z_hbm.at[pl.ds(0, RPS_LAST)],
                agg_sh.at[pl.ds((NS - 1) * RPS, RPS_LAST)],
            )

        plsc.subcore_barrier()

        last = NCHUNK - 1

        def gather(j, buf, sem):
            pltpu.async_copy(
                x_hbm.at[src_v.at[pl.ds(j * CHUNK, CHUNK)]], buf, sem
            )

        gather(0, rows0_v, sem0)
        gather(1, rows1_v, sem1)

        def body(i, carry):
            j0 = 2 * i
            pltpu.make_async_copy(x_hbm.at[pl.ds(0, CHUNK)], rows0_v, sem0).wait()
            pltpu.sync_copy(rows0_v, agg_sh.at[dst_v.at[j0]], add=True)
            gather(jnp.minimum(j0 + 2, last), rows0_v, sem0)
            pltpu.make_async_copy(x_hbm.at[pl.ds(0, CHUNK)], rows1_v, sem1).wait()
            pltpu.sync_copy(rows1_v, agg_sh.at[dst_v.at[j0 + 1]], add=True)
            gather(jnp.minimum(j0 + 3, last), rows1_v, sem1)
            return carry

        lax.fori_loop(0, NCH2, body, 0)
        pltpu.make_async_copy(x_hbm.at[pl.ds(0, CHUNK)], rows0_v, sem0).wait()
        pltpu.make_async_copy(x_hbm.at[pl.ds(0, CHUNK)], rows1_v, sem1).wait()
        plsc.subcore_barrier()

        @pl.when(s < NS - 1)
        def _():
            pltpu.sync_copy(
                agg_sh.at[pl.ds(r0, RPS)], out_hbm.at[c, pl.ds(r0, RPS)]
            )

        @pl.when(s == NS - 1)
        def _():
            pltpu.sync_copy(
                agg_sh.at[pl.ds((NS - 1) * RPS, RPS_LAST)],
                out_hbm.at[c, pl.ds((NS - 1) * RPS, RPS_LAST)],
            )

    return k(xp, src_flat, dst_r, zeros)


def _tc_finish(x, parts, w2d, W1, b1, W2, b2):
    def body(x_ref, p_ref, w_ref, w1_ref, b1_ref, w2_ref, b2_ref, out_ref):
        h = x_ref[...] + p_ref[0] + p_ref[1]
        t = jnp.dot(h, w1_ref[...].T, preferred_element_type=jnp.float32)
        t = jnp.maximum(t + b1_ref[...], 0.0)
        wv = w_ref[...]
        v = jnp.sum(t * wv, axis=0, keepdims=True)
        sw = jnp.sum(wv)
        out = jnp.dot(v, w2_ref[...].T, preferred_element_type=jnp.float32)
        out_ref[...] = out + sw * b2_ref[...]

    return pl.pallas_call(
        body,
        out_shape=jax.ShapeDtypeStruct((1, D), jnp.float32),
    )(x, parts, w2d, W1, b1, W2, b2)


def kernel(x, edge_index, weights, W1, b1, W2, b2):
    pad = EPW_PAD - EPW
    src = edge_index[0].reshape(NW, EPW)
    dst = edge_index[1].reshape(NW, EPW)
    src_p = jnp.pad(src, ((0, 0), (0, pad)), constant_values=N).reshape(-1)
    dst_p = jnp.pad(dst, ((0, 0), (0, pad))).reshape(NW, NCHUNK, CHUNK)
    xp = jnp.pad(x, ((0, NPAD - N), (0, 0)))
    zeros = jnp.zeros((RPS_LAST, D), jnp.float32)
    parts = _sc_aggregate(xp, src_p, dst_p, zeros)
    out = _tc_finish(x, parts, weights.reshape(N, 1), W1, b1, W2, b2)
    return out.reshape(1, 1, D)


# speedup vs baseline: 1.3822x; 1.3822x over previous
import functools

import jax
import jax.numpy as jnp
from jax import lax
from jax.experimental import pallas as pl
from jax.experimental.pallas import tpu as pltpu
from jax.experimental.pallas import tpu_sc as plsc

N = 10000
E = 320000
D = 128
NC, NS = 2, 16
NW = NC * NS
EPW = E // NW
CHUNK = 104
NCHUNK = 98
EPW_PAD = NCHUNK * CHUNK
NCH2 = NCHUNK // 2
NPAD = N + 8
RPS = 624
RPS_LAST = N - (NS - 1) * RPS


def _sc_aggregate(xp, src_flat, dst_r, zeros):
    mesh = plsc.VectorSubcoreMesh(core_axis_name="c", subcore_axis_name="s")

    @functools.partial(
        pl.kernel,
        out_type=jax.ShapeDtypeStruct((NC, N, D), jnp.float32),
        mesh=mesh,
        scratch_types=[
            pltpu.VMEM((EPW_PAD,), jnp.int32),
            pltpu.VMEM((NCHUNK, CHUNK), jnp.int32),
            pltpu.VMEM((CHUNK, D), jnp.float32),
            pltpu.VMEM((CHUNK, D), jnp.float32),
            pltpu.VMEM_SHARED((N, D), jnp.float32),
            pltpu.SemaphoreType.DMA,
            pltpu.SemaphoreType.DMA,
        ],
    )
    def k(x_hbm, src_hbm, dst_hbm, z_hbm, out_hbm, src_v, dst_v,
          rows0_v, rows1_v, agg_sh, sem0, sem1):
        c = lax.axis_index("c")
        s = lax.axis_index("s")
        wid = c * NS + s
        pltpu.sync_copy(src_hbm.at[pl.ds(wid * EPW_PAD, EPW_PAD)], src_v)
        pltpu.sync_copy(dst_hbm.at[wid], dst_v)
        r0 = s * RPS

        @pl.when(s < NS - 1)
        def _():
            pltpu.sync_copy(z_hbm.at[pl.ds(0, RPS)], agg_sh.at[pl.ds(r0, RPS)])

        @pl.when(s == NS - 1)
        def _():
            pltpu.sync_copy(
                z_hbm.at[pl.ds(0, RPS_LAST)],
                agg_sh.at[pl.ds((NS - 1) * RPS, RPS_LAST)],
            )

        plsc.subcore_barrier()

        last = NCHUNK - 1

        def gather(j, buf, sem):
            pltpu.async_copy(
                x_hbm.at[src_v.at[pl.ds(j * CHUNK, CHUNK)]], buf, sem
            )

        def body(j, carry):
            pltpu.async_copy(
                x_hbm.at[src_v.at[pl.ds(j * CHUNK, CHUNK)]], rows0_v, sem0
            ).wait()
            pltpu.sync_copy(rows0_v, agg_sh.at[dst_v.at[j]], add=True)
            return carry

        lax.fori_loop(0, NCHUNK, body, 0)
        plsc.subcore_barrier()

        @pl.when(s < NS - 1)
        def _():
            pltpu.sync_copy(
                agg_sh.at[pl.ds(r0, RPS)], out_hbm.at[c, pl.ds(r0, RPS)]
            )

        @pl.when(s == NS - 1)
        def _():
            pltpu.sync_copy(
                agg_sh.at[pl.ds((NS - 1) * RPS, RPS_LAST)],
                out_hbm.at[c, pl.ds((NS - 1) * RPS, RPS_LAST)],
            )

    return k(xp, src_flat, dst_r, zeros)


def _tc_finish(x, parts, w2d, W1, b1, W2, b2):
    def body(x_ref, p_ref, w_ref, w1_ref, b1_ref, w2_ref, b2_ref, out_ref):
        h = x_ref[...] + p_ref[0] + p_ref[1]
        t = jnp.dot(h, w1_ref[...].T, preferred_element_type=jnp.float32)
        t = jnp.maximum(t + b1_ref[...], 0.0)
        wv = w_ref[...]
        v = jnp.sum(t * wv, axis=0, keepdims=True)
        sw = jnp.sum(wv)
        out = jnp.dot(v, w2_ref[...].T, preferred_element_type=jnp.float32)
        out_ref[...] = out + sw * b2_ref[...]

    return pl.pallas_call(
        body,
        out_shape=jax.ShapeDtypeStruct((1, D), jnp.float32),
    )(x, parts, w2d, W1, b1, W2, b2)


def kernel(x, edge_index, weights, W1, b1, W2, b2):
    pad = EPW_PAD - EPW
    src = edge_index[0].reshape(NW, EPW)
    dst = edge_index[1].reshape(NW, EPW)
    src_p = jnp.pad(src, ((0, 0), (0, pad)), constant_values=N).reshape(-1)
    dst_p = jnp.pad(dst, ((0, 0), (0, pad))).reshape(NW, NCHUNK, CHUNK)
    xp = jnp.pad(x, ((0, NPAD - N), (0, 0)))
    zeros = jnp.zeros((RPS_LAST, D), jnp.float32)
    parts = _sc_aggregate(xp, src_p, dst_p, zeros)
    out = _tc_finish(x, parts, weights.reshape(N, 1), W1, b1, W2, b2)
    return out.reshape(1, 1, D)
